# Initial kernel scaffold; baseline (speedup 1.0000x reference)
import jax
import jax.numpy as jnp
from jax.experimental import pallas as pl


def kernel(src, idx, tgt):
    raise NotImplementedError("write your pallas kernel here")


# speedup vs baseline: 1.8047x; 1.8047x over previous
import jax
import jax.numpy as jnp
from jax.experimental import pallas as pl
from jax.experimental.pallas import tpu as pltpu

_N = 16384
_M = 16384
_BS = 2048
_TC = 4096
_MAXD = 1.0


def _nn_loss_kernel(src_ref, tgtT_ref, out_ref, b_ref):
    i = pl.program_id(0)

    @pl.when(i == 0)
    def _():
        t = tgtT_ref[...]
        t2 = jnp.sum(t * t, axis=0, keepdims=True)
        h = t2.astype(jnp.bfloat16)
        r1 = t2 - h.astype(jnp.float32)
        m = r1.astype(jnp.bfloat16)
        lo = (r1 - m.astype(jnp.float32)).astype(jnp.bfloat16)
        z = jnp.zeros((1, _M), jnp.bfloat16)
        b_ref[...] = jnp.concatenate(
            [t.astype(jnp.bfloat16), h, m, lo, z, z], axis=0)

    s = src_ref[...]
    s2 = jnp.sum(s * s, axis=1, keepdims=True)
    sm2 = (-2.0 * s).astype(jnp.bfloat16)
    a_mat = jnp.concatenate(
        [sm2,
         jnp.ones((_BS, 3), jnp.bfloat16),
         jnp.zeros((_BS, 2), jnp.bfloat16)], axis=1)

    def matmul(k):
        return jax.lax.dot_general(
            a_mat, b_ref[:, pl.ds(k * _TC, _TC)],
            (((1,), (0,)), ((), ())),
            preferred_element_type=jnp.float32,
        )

    acc = jnp.full((_BS, 128), jnp.inf, dtype=jnp.float32)
    for k in range(_M // _TC):
        d = matmul(k)
        for g in range(_TC // 128):
            acc = jnp.minimum(acc, d[:, g * 128:(g + 1) * 128])
    mins = jnp.min(acc, axis=1, keepdims=True) + s2
    mins = jnp.maximum(mins, 0.0)
    clamped = jnp.where(mins > _MAXD, 0.0, mins)
    psum = jnp.sum(clamped) * (1.0 / _N)

    @pl.when(i == 0)
    def _():
        out_ref[0, 0] = 0.0

    out_ref[0, 0] += psum


def kernel(src, idx, tgt):
    del idx
    tgtT = tgt.T
    out = pl.pallas_call(
        _nn_loss_kernel,
        grid=(_N // _BS,),
        in_specs=[
            pl.BlockSpec((_BS, 3), lambda i: (i, 0)),
            pl.BlockSpec((3, _M), lambda i: (0, 0)),
        ],
        out_specs=pl.BlockSpec(memory_space=pltpu.SMEM),
        out_shape=jax.ShapeDtypeStruct((1, 1), jnp.float32),
        scratch_shapes=[pltpu.VMEM((8, _M), jnp.bfloat16)],
    )(src, tgtT)
    return out[0, 0]
